# Initial kernel scaffold; baseline (speedup 1.0000x reference)
import jax
import jax.numpy as jnp
from jax.experimental import pallas as pl


def kernel(input, edge_index, edge_weight, weight, bias):
    raise NotImplementedError("write your pallas kernel here")


# speedup vs baseline: 4.0514x; 4.0514x over previous
import functools

import jax
import jax.numpy as jnp
from jax import lax
from jax.experimental import pallas as pl
from jax.experimental.pallas import tpu as pltpu
from jax.experimental.pallas import tpu_sc as plsc

NC, NS, LANES = 2, 16, 16
CHUNK = 80


def _sc_aggregate(x, cols, rows, ew):
    n, d = x.shape
    e = ew.shape[0]
    nw = NC * NS
    e_per_tile = e // nw
    n_chunks = e_per_tile // CHUNK
    rows_per_tile = n // NS
    wb = rows_per_tile // 5
    nwb = rows_per_tile // wb
    mesh = plsc.VectorSubcoreMesh(core_axis_name="c", subcore_axis_name="s",
                                  num_cores=NC, num_subcores=NS)

    @functools.partial(
        pl.kernel,
        out_type=jax.ShapeDtypeStruct((NC, n, d), jnp.float32),
        mesh=mesh,
        scratch_types=[
            pltpu.VMEM_SHARED((n, d), jnp.float32),
            pltpu.VMEM((CHUNK,), jnp.int32),
            pltpu.VMEM((CHUNK,), jnp.int32),
            pltpu.VMEM((CHUNK,), jnp.float32),
            pltpu.VMEM((CHUNK, d), jnp.float32),
            pltpu.VMEM((wb, d), jnp.float32),
            pltpu.SemaphoreType.DMA,
        ],
    )
    def agg(x_hbm, cols_hbm, rows_hbm, ew_hbm, out_hbm,
            acc, colb, rowb, ewb, gb, stage, sem):
        c = lax.axis_index("c")
        s = lax.axis_index("s")
        zero16 = jnp.zeros((LANES,), jnp.float32)

        def zrow(i, carry):
            for r in range(d // LANES):
                stage[i, pl.ds(r * LANES, LANES)] = zero16
            return carry

        lax.fori_loop(0, wb, zrow, 0)
        row0 = s * rows_per_tile
        for t in range(nwb):
            pltpu.sync_copy(stage, acc.at[pl.ds(row0 + t * wb, wb)])
        plsc.subcore_barrier()

        tile_base = (c * NS + s) * e_per_tile

        def chunk_body(j, carry):
            base = tile_base + j * CHUNK
            pltpu.sync_copy(cols_hbm.at[pl.ds(base, CHUNK)], colb)
            pltpu.sync_copy(ew_hbm.at[pl.ds(base, CHUNK)], ewb)
            pltpu.sync_copy(rows_hbm.at[pl.ds(base, CHUNK)], rowb)
            pltpu.async_copy(x_hbm.at[colb], gb, sem).wait()

            def group_body(g, icarry):
                wv = ewb[pl.ds(g * LANES, LANES)]
                for lane in range(LANES):
                    w = wv[lane]
                    i = g * LANES + lane
                    for r in range(d // LANES):
                        sl = pl.ds(r * LANES, LANES)
                        gb[i, sl] = gb[i, sl] * w
                return icarry

            lax.fori_loop(0, CHUNK // LANES, group_body, 0)
            pltpu.sync_copy(gb, acc.at[rowb], add=True)
            return carry

        lax.fori_loop(0, n_chunks, chunk_body, 0)
        plsc.subcore_barrier()

        for t in range(nwb):
            r0 = row0 + t * wb
            pltpu.sync_copy(acc.at[pl.ds(r0, wb)], stage)
            pltpu.sync_copy(stage, out_hbm.at[c, pl.ds(r0, wb)])

    return agg(x, cols, rows, ew)


def _tc_combine_matmul(p0, p1, w, b):
    n, d = p0.shape
    blk = 1024

    def mm(p0_ref, p1_ref, w_ref, b_ref, o_ref):
        acc = p0_ref[...] + p1_ref[...]
        o_ref[...] = (
            jnp.dot(acc, w_ref[...], preferred_element_type=jnp.float32)
            + b_ref[...]
        )

    return pl.pallas_call(
        mm,
        grid=(n // blk,),
        in_specs=[
            pl.BlockSpec((blk, d), lambda i: (i, 0)),
            pl.BlockSpec((blk, d), lambda i: (i, 0)),
            pl.BlockSpec((d, d), lambda i: (0, 0)),
            pl.BlockSpec((1, d), lambda i: (0, 0)),
        ],
        out_specs=pl.BlockSpec((blk, d), lambda i: (i, 0)),
        out_shape=jax.ShapeDtypeStruct((n, d), jnp.float32),
    )(p0, p1, w, b)


def kernel(input, edge_index, edge_weight, weight, bias):
    ei = edge_index.astype(jnp.int32)
    rows, cols = ei[0], ei[1]
    ew = edge_weight
    step = NC * NS * CHUNK
    epad = (-ew.shape[0]) % step
    if epad:
        zi = jnp.zeros((epad,), jnp.int32)
        cols = jnp.concatenate([cols, zi])
        rows = jnp.concatenate([rows, zi])
        ew = jnp.concatenate([ew, jnp.zeros((epad,), ew.dtype)])
    n, d = input.shape
    npad = (-n) % (NS * 64)
    x = input
    if npad:
        x = jnp.concatenate([x, jnp.zeros((npad, d), x.dtype)], axis=0)
    partials = _sc_aggregate(x, cols, rows, ew)
    out = _tc_combine_matmul(partials[0], partials[1], weight,
                             bias.reshape(1, -1))
    return out[:n]
